# Initial kernel scaffold; baseline (speedup 1.0000x reference)
import jax
import jax.numpy as jnp
from jax.experimental import pallas as pl


def kernel(feature_indices, feature_values, edge_indices, edge_weights, W1, b1, W2, b2):
    raise NotImplementedError("write your pallas kernel here")


# speedup vs baseline: 4.4596x; 4.4596x over previous
import functools

import jax
import jax.numpy as jnp
from jax import lax
from jax.experimental import pallas as pl
from jax.experimental.pallas import tpu as pltpu
from jax.experimental.pallas import tpu_sc as plsc

N_NODES = 10000
HIDDEN = 64
N_LABELS = 64
ALPHA = 0.1
ITERATIONS_ = 10

NC = 2
NS = 16
NW = NC * NS
BLK = 128
NPAD = 10240
ROWS_PER_TILE = NPAD // NS
LANES = 16
CGRP = HIDDEN // LANES


def _make_seg_kernel(e_pad):
    ew = e_pad // NW
    nblk = ew // BLK
    mesh = plsc.VectorSubcoreMesh(core_axis_name="c", subcore_axis_name="s")

    @functools.partial(
        pl.kernel,
        mesh=mesh,
        compiler_params=pltpu.CompilerParams(
            needs_layout_passes=False, use_tc_tiling_on_sc=False),
        out_type=jax.ShapeDtypeStruct((NC, NPAD, HIDDEN), jnp.float32),
        scratch_types=[
            pltpu.VMEM_SHARED((NPAD, HIDDEN), jnp.float32),
            pltpu.VMEM((ew,), jnp.float32),
            pltpu.VMEM((BLK,), jnp.int32),
            pltpu.VMEM((BLK,), jnp.int32),
            pltpu.VMEM((BLK, HIDDEN), jnp.float32),
            pltpu.SemaphoreType.DMA,
        ],
    )
    def seg(table, gidx, w, sidx, zrows, out, acc, w_v, gidx_v, sidx_v,
            rows_v, sem):
        cid = lax.axis_index("c")
        sid = lax.axis_index("s")
        wid = cid * NS + sid
        wbase = wid * ew
        rslice = pl.ds(sid * ROWS_PER_TILE, ROWS_PER_TILE)
        pltpu.sync_copy(zrows, acc.at[rslice])
        pltpu.sync_copy(w.at[pl.ds(wbase, ew)], w_v)
        plsc.subcore_barrier()

        def blk_body(b, carry):
            base = b * BLK
            pltpu.sync_copy(gidx.at[pl.ds(wbase + base, BLK)], gidx_v)
            pltpu.sync_copy(sidx.at[pl.ds(wbase + base, BLK)], sidx_v)
            pltpu.async_copy(table.at[gidx_v], rows_v, sem).wait()

            def e_body(e, c2):
                wv = plsc.load_gather(
                    w_v, [jnp.broadcast_to(base + e, (LANES,)).astype(jnp.int32)])
                for g in range(CGRP):
                    s = pl.ds(g * LANES, LANES)
                    rows_v[e, s] = rows_v[e, s] * wv
                return c2

            lax.fori_loop(0, BLK, e_body, 0)
            pltpu.sync_copy(rows_v, acc.at[sidx_v], add=True)
            return carry

        lax.fori_loop(0, nblk, blk_body, 0)
        plsc.subcore_barrier()
        pltpu.sync_copy(acc.at[rslice], out.at[cid, rslice])

    return seg


_E1_PAD = ((500000 + NW * BLK - 1) // (NW * BLK)) * (NW * BLK)
_E2_PAD = ((320000 + NW * BLK - 1) // (NW * BLK)) * (NW * BLK)
_SEG1 = _make_seg_kernel(_E1_PAD)
_SEG2 = _make_seg_kernel(_E2_PAD)


def _seg_partials(kern, e_pad, table, gidx, w, sidx, zrows):
    pad = e_pad - gidx.shape[0]
    return kern(
        table,
        jnp.pad(gidx, (0, pad)),
        jnp.pad(w, (0, pad)),
        jnp.pad(sidx, (0, pad)),
        zrows,
    )


_R = 2000


def _mlp_body(p_ref, b1_ref, w2_ref, b2_ref, o_ref):
    h = jnp.maximum(p_ref[0] + p_ref[1] + b1_ref[...], 0.0)
    o_ref[...] = (
        jnp.dot(h, w2_ref[...], preferred_element_type=jnp.float32)
        + b2_ref[...]
    )


def _mlp(p, b1, W2, b2):
    return pl.pallas_call(
        _mlp_body,
        grid=(N_NODES // _R,),
        in_specs=[
            pl.BlockSpec((NC, _R, HIDDEN), lambda i: (0, i, 0)),
            pl.BlockSpec((1, HIDDEN), lambda i: (0, 0)),
            pl.BlockSpec((HIDDEN, N_LABELS), lambda i: (0, 0)),
            pl.BlockSpec((1, N_LABELS), lambda i: (0, 0)),
        ],
        out_specs=pl.BlockSpec((_R, N_LABELS), lambda i: (i, 0)),
        out_shape=jax.ShapeDtypeStruct((N_NODES, N_LABELS), jnp.float32),
    )(p, b1.reshape(1, HIDDEN), W2, b2.reshape(1, N_LABELS))


def _combine_body(q_ref, h2_ref, o_ref):
    o_ref[...] = (1.0 - ALPHA) * (q_ref[0] + q_ref[1]) + ALPHA * h2_ref[...]


def _combine(q, h2):
    return pl.pallas_call(
        _combine_body,
        grid=(N_NODES // _R,),
        in_specs=[
            pl.BlockSpec((NC, _R, N_LABELS), lambda i: (0, i, 0)),
            pl.BlockSpec((_R, N_LABELS), lambda i: (i, 0)),
        ],
        out_specs=pl.BlockSpec((_R, N_LABELS), lambda i: (i, 0)),
        out_shape=jax.ShapeDtypeStruct((N_NODES, N_LABELS), jnp.float32),
    )(q, h2)


def _combine_ls_body(q_ref, h2_ref, o_ref):
    t = (1.0 - ALPHA) * (q_ref[0] + q_ref[1]) + ALPHA * h2_ref[...]
    m = jnp.max(t, axis=1, keepdims=True)
    e = jnp.exp(t - m)
    o_ref[...] = t - m - jnp.log(jnp.sum(e, axis=1, keepdims=True))


def _combine_ls(q, h2):
    return pl.pallas_call(
        _combine_ls_body,
        grid=(N_NODES // _R,),
        in_specs=[
            pl.BlockSpec((NC, _R, N_LABELS), lambda i: (0, i, 0)),
            pl.BlockSpec((_R, N_LABELS), lambda i: (i, 0)),
        ],
        out_specs=pl.BlockSpec((_R, N_LABELS), lambda i: (i, 0)),
        out_shape=jax.ShapeDtypeStruct((N_NODES, N_LABELS), jnp.float32),
    )(q, h2)


def kernel(feature_indices, feature_values, edge_indices, edge_weights,
           W1, b1, W2, b2):
    zrows = jnp.zeros((ROWS_PER_TILE, HIDDEN), jnp.float32)
    p = _seg_partials(_SEG1, _E1_PAD, W1, feature_indices[1],
                      feature_values, feature_indices[0], zrows)
    h2 = _mlp(p, b1, W2, b2)
    loc = h2
    out = None
    for i in range(ITERATIONS_):
        q = _seg_partials(_SEG2, _E2_PAD, loc, edge_indices[1],
                          edge_weights, edge_indices[0], zrows)
        if i < ITERATIONS_ - 1:
            loc = _combine(q, h2)
        else:
            out = _combine_ls(q, h2)
    return out


# speedup vs baseline: 6.0342x; 1.3531x over previous
import functools

import jax
import jax.numpy as jnp
from jax import lax
from jax.experimental import pallas as pl
from jax.experimental.pallas import tpu as pltpu
from jax.experimental.pallas import tpu_sc as plsc

N_NODES = 10000
HIDDEN = 64
N_LABELS = 64
ALPHA = 0.1
ITERATIONS_ = 10

NC = 2
NS = 16
NW = NC * NS
BLK = 128
NPAD = 10240
ROWS_PER_TILE = NPAD // NS
LANES = 16
CGRP = HIDDEN // LANES


NBUF = 4
LOOK = 2


def _make_seg_kernel(e_pad):
    ew = e_pad // NW
    nblk = ew // BLK
    assert nblk % NBUF == 0
    mesh = plsc.VectorSubcoreMesh(core_axis_name="c", subcore_axis_name="s")

    @functools.partial(
        pl.kernel,
        mesh=mesh,
        compiler_params=pltpu.CompilerParams(
            needs_layout_passes=False, use_tc_tiling_on_sc=False),
        out_type=jax.ShapeDtypeStruct((NC, NPAD, HIDDEN), jnp.float32),
        scratch_types=[
            pltpu.VMEM_SHARED((NPAD, HIDDEN), jnp.float32),
            pltpu.VMEM((ew,), jnp.float32),
            pltpu.VMEM((nblk, BLK), jnp.int32),
            pltpu.VMEM((nblk, BLK), jnp.int32),
            pltpu.VMEM((NBUF, BLK, HIDDEN), jnp.float32),
        ]
        + [pltpu.SemaphoreType.DMA] * NBUF
        + [pltpu.SemaphoreType.DMA] * NBUF,
    )
    def seg(table, gidx, w, sidx, zrows, out, acc, w_v, gidx_v, sidx_v,
            rows_v, *sems):
        sem_g = sems[:NBUF]
        sem_s = sems[NBUF:]
        cid = lax.axis_index("c")
        sid = lax.axis_index("s")
        wid = cid * NS + sid
        rslice = pl.ds(sid * ROWS_PER_TILE, ROWS_PER_TILE)
        pltpu.sync_copy(zrows, acc.at[rslice])
        pltpu.sync_copy(w.at[wid], w_v)
        pltpu.sync_copy(gidx.at[wid], gidx_v)
        pltpu.sync_copy(sidx.at[wid], sidx_v)
        plsc.subcore_barrier()

        def fire_gather(b, j):
            pltpu.async_copy(table.at[gidx_v.at[b]], rows_v.at[j], sem_g[j])

        def wait_gather(j):
            pltpu.make_async_copy(
                table.at[gidx_v.at[0]], rows_v.at[j], sem_g[j]).wait()

        def fire_scatter(b, j):
            pltpu.async_copy(
                rows_v.at[j], acc.at[sidx_v.at[b]], sem_s[j], add=True)

        def wait_scatter(j):
            pltpu.make_async_copy(
                rows_v.at[j], acc.at[sidx_v.at[0]], sem_s[j]).wait()

        for j in range(LOOK):
            fire_gather(j, j)

        def super_body(s8, carry):
            b0 = s8 * NBUF
            for j in range(NBUF):
                b = b0 + j
                wait_gather(j)

                def e_body(e, c2):
                    wv = plsc.load_gather(
                        w_v,
                        [jnp.broadcast_to(b * BLK + e, (LANES,)).astype(
                            jnp.int32)])
                    for g in range(CGRP):
                        sl = pl.ds(g * LANES, LANES)
                        rows_v[j, e, sl] = rows_v[j, e, sl] * wv
                    return c2

                lax.fori_loop(0, BLK, e_body, 0, unroll=4)
                fire_scatter(b, j)
                bf = b + LOOK
                jf = (j + LOOK) % NBUF

                @pl.when(bf < nblk)
                def _():
                    @pl.when(b >= LOOK)
                    def _():
                        wait_scatter(jf)
                    fire_gather(bf, jf)

            return carry

        lax.fori_loop(0, nblk // NBUF, super_body, 0)
        for j in range(NBUF):
            wait_scatter(j)
        plsc.subcore_barrier()
        pltpu.sync_copy(acc.at[rslice], out.at[cid, rslice])

    return seg


_GRAN = NW * BLK * NBUF
_E1_PAD = ((500000 + _GRAN - 1) // _GRAN) * _GRAN
_E2_PAD = ((320000 + _GRAN - 1) // _GRAN) * _GRAN
_SEG1 = _make_seg_kernel(_E1_PAD)
_SEG2 = _make_seg_kernel(_E2_PAD)


def _seg_partials(kern, e_pad, table, gidx, w, sidx, zrows):
    pad = e_pad - gidx.shape[0]
    ew = e_pad // NW
    nblk = ew // BLK
    return kern(
        table,
        jnp.pad(gidx, (0, pad)).reshape(NW, nblk, BLK),
        jnp.pad(w, (0, pad)).reshape(NW, ew),
        jnp.pad(sidx, (0, pad)).reshape(NW, nblk, BLK),
        zrows,
    )


_R = 2000


def _mlp_body(p_ref, b1_ref, w2_ref, b2_ref, o_ref):
    h = jnp.maximum(p_ref[0] + p_ref[1] + b1_ref[...], 0.0)
    o_ref[...] = (
        jnp.dot(h, w2_ref[...], preferred_element_type=jnp.float32)
        + b2_ref[...]
    )


def _mlp(p, b1, W2, b2):
    return pl.pallas_call(
        _mlp_body,
        grid=(N_NODES // _R,),
        in_specs=[
            pl.BlockSpec((NC, _R, HIDDEN), lambda i: (0, i, 0)),
            pl.BlockSpec((1, HIDDEN), lambda i: (0, 0)),
            pl.BlockSpec((HIDDEN, N_LABELS), lambda i: (0, 0)),
            pl.BlockSpec((1, N_LABELS), lambda i: (0, 0)),
        ],
        out_specs=pl.BlockSpec((_R, N_LABELS), lambda i: (i, 0)),
        out_shape=jax.ShapeDtypeStruct((N_NODES, N_LABELS), jnp.float32),
    )(p, b1.reshape(1, HIDDEN), W2, b2.reshape(1, N_LABELS))


def _combine_body(q_ref, h2_ref, o_ref):
    o_ref[...] = (1.0 - ALPHA) * (q_ref[0] + q_ref[1]) + ALPHA * h2_ref[...]


def _combine(q, h2):
    return pl.pallas_call(
        _combine_body,
        grid=(N_NODES // _R,),
        in_specs=[
            pl.BlockSpec((NC, _R, N_LABELS), lambda i: (0, i, 0)),
            pl.BlockSpec((_R, N_LABELS), lambda i: (i, 0)),
        ],
        out_specs=pl.BlockSpec((_R, N_LABELS), lambda i: (i, 0)),
        out_shape=jax.ShapeDtypeStruct((N_NODES, N_LABELS), jnp.float32),
    )(q, h2)


def _combine_ls_body(q_ref, h2_ref, o_ref):
    t = (1.0 - ALPHA) * (q_ref[0] + q_ref[1]) + ALPHA * h2_ref[...]
    m = jnp.max(t, axis=1, keepdims=True)
    e = jnp.exp(t - m)
    o_ref[...] = t - m - jnp.log(jnp.sum(e, axis=1, keepdims=True))


def _combine_ls(q, h2):
    return pl.pallas_call(
        _combine_ls_body,
        grid=(N_NODES // _R,),
        in_specs=[
            pl.BlockSpec((NC, _R, N_LABELS), lambda i: (0, i, 0)),
            pl.BlockSpec((_R, N_LABELS), lambda i: (i, 0)),
        ],
        out_specs=pl.BlockSpec((_R, N_LABELS), lambda i: (i, 0)),
        out_shape=jax.ShapeDtypeStruct((N_NODES, N_LABELS), jnp.float32),
    )(q, h2)


def kernel(feature_indices, feature_values, edge_indices, edge_weights,
           W1, b1, W2, b2):
    zrows = jnp.zeros((ROWS_PER_TILE, HIDDEN), jnp.float32)
    p = _seg_partials(_SEG1, _E1_PAD, W1, feature_indices[1],
                      feature_values, feature_indices[0], zrows)
    h2 = _mlp(p, b1, W2, b2)
    loc = h2
    out = None
    for i in range(ITERATIONS_):
        q = _seg_partials(_SEG2, _E2_PAD, loc, edge_indices[1],
                          edge_weights, edge_indices[0], zrows)
        if i < ITERATIONS_ - 1:
            loc = _combine(q, h2)
        else:
            out = _combine_ls(q, h2)
    return out
